# baseline (speedup 1.0000x reference)
import functools

import jax
import jax.numpy as jnp
from jax import lax
from jax.experimental import pallas as pl
from jax.experimental.pallas import tpu as pltpu
from jax.experimental.pallas import tpu_sc as plsc

B, L, V, D = 4096, 200, 1000000, 32
NC, NS = 2, 16
NW = NC * NS
RPW = B // NW
LN = 16


def _body(wid_hbm, lens_hbm, w_hbm, out_hbm,
          wid_v, lensb_v, buckb_v, rows0, rows1, out_v, w0_v, lens_sc,
          sem0, sem1):
    cid = lax.axis_index("c")
    sid = lax.axis_index("s")
    w = cid * NS + sid
    base = w * RPW

    pltpu.sync_copy(wid_hbm.at[pl.ds(base, RPW), :], wid_v.at[:, :])
    pltpu.sync_copy(w_hbm.at[pl.ds(0, 1), :], w0_v)
    pltpu.sync_copy(lens_hbm.at[pl.ds(base, RPW)], lens_sc)

    lane = lax.iota(jnp.int32, LN)
    iotaf = lane.astype(jnp.float32)

    def prep_group(g, _):
        lv = lens_sc[pl.ds(g * LN, LN)]

        def prep_row(r2, _):
            r = g * LN + r2
            len_i = jnp.max(jnp.where(lane == r2, lv, 0))
            lenf = len_i.astype(jnp.float32)
            bucket = jnp.minimum((len_i + 31) // 32 * 32, L)
            lensb_v[r, :] = jnp.full((LN,), 1.0, jnp.float32) * lenf
            buckb_v[r, :] = jnp.full((LN,), 1.0, jnp.float32) * bucket.astype(jnp.float32)
            lfb = jnp.full((LN,), 1.0, jnp.float32) * lenf
            for c in range(13):
                off = min(c * LN, L - LN)
                ids = wid_v[r, pl.ds(off, LN)]
                posf = iotaf + float(off)
                wid_v[r, pl.ds(off, LN)] = jnp.where(posf < lfb, ids, 0)
            return 0

        lax.fori_loop(0, LN, prep_row, 0)
        return 0

    lax.fori_loop(0, RPW // LN, prep_group, 0)

    def nbuckets(r):
        return (jnp.max(buckb_v[r, :]).astype(jnp.int32) + 31) // 32

    def fire(r, rows_buf, sem):
        nb = nbuckets(r)

        @pl.when(nb <= 4)
        def _():
            for k in (1, 2, 3, 4):
                @pl.when(nb == k)
                def _():
                    pltpu.async_copy(
                        w_hbm.at[wid_v.at[r, pl.ds(0, 32 * k)]],
                        rows_buf.at[pl.ds(0, 32 * k), :], sem)

        @pl.when(nb > 4)
        def _():
            pltpu.async_copy(
                w_hbm.at[wid_v.at[r, pl.ds(0, 128)]],
                rows_buf.at[pl.ds(0, 128), :], sem)

            @pl.when(nb == 5)
            def _():
                pltpu.async_copy(
                    w_hbm.at[wid_v.at[r, pl.ds(128, 32)]],
                    rows_buf.at[pl.ds(128, 32), :], sem)

            @pl.when(nb >= 6)
            def _():
                pltpu.async_copy(
                    w_hbm.at[wid_v.at[r, pl.ds(128, 72)]],
                    rows_buf.at[pl.ds(128, 72), :], sem)

    def drain(r, rows_buf, sem):
        nb = nbuckets(r)

        @pl.when(nb <= 4)
        def _():
            for k in (1, 2, 3, 4):
                @pl.when(nb == k)
                def _():
                    pltpu.make_async_copy(
                        w_hbm.at[wid_v.at[r, pl.ds(0, 32 * k)]],
                        rows_buf.at[pl.ds(0, 32 * k), :], sem).wait()

        @pl.when(nb > 4)
        def _():
            pltpu.make_async_copy(
                w_hbm.at[wid_v.at[r, pl.ds(0, 128)]],
                rows_buf.at[pl.ds(0, 128), :], sem).wait()

            @pl.when(nb == 5)
            def _():
                pltpu.make_async_copy(
                    w_hbm.at[wid_v.at[r, pl.ds(128, 32)]],
                    rows_buf.at[pl.ds(128, 32), :], sem).wait()

            @pl.when(nb >= 6)
            def _():
                pltpu.make_async_copy(
                    w_hbm.at[wid_v.at[r, pl.ds(128, 72)]],
                    rows_buf.at[pl.ds(128, 72), :], sem).wait()

    def accum(r, rows_buf):
        lfb = lensb_v[r, :]
        bfb = buckb_v[r, :]
        bk = jnp.max(bfb).astype(jnp.int32)
        zero = jnp.zeros((LN,), jnp.float32)

        def block_body(c, accs):
            accs = list(accs)
            b32 = c * 32
            for u in range(32):
                accs[2 * (u % 4)] = accs[2 * (u % 4)] + rows_buf[b32 + u, pl.ds(0, LN)]
                accs[2 * (u % 4) + 1] = accs[2 * (u % 4) + 1] + rows_buf[b32 + u, pl.ds(LN, LN)]
            return tuple(accs)

        accs = lax.fori_loop(0, bk // 32, block_body, (zero,) * 8)
        accs = list(accs)

        @pl.when(bk == L)
        def _():
            a = [rows_buf[192 + u, pl.ds(0, LN)] for u in range(8)]
            b = [rows_buf[192 + u, pl.ds(LN, LN)] for u in range(8)]
            t0 = ((a[0] + a[1]) + (a[2] + a[3])) + ((a[4] + a[5]) + (a[6] + a[7]))
            t1 = ((b[0] + b[1]) + (b[2] + b[3])) + ((b[4] + b[5]) + (b[6] + b[7]))
            out_v[r, pl.ds(0, LN)] = t0
            out_v[r, pl.ds(LN, LN)] = t1

        @pl.when(bk < L)
        def _():
            out_v[r, pl.ds(0, LN)] = zero
            out_v[r, pl.ds(LN, LN)] = zero

        acc0 = ((accs[0] + accs[2]) + (accs[4] + accs[6])) + out_v[r, pl.ds(0, LN)]
        acc1 = ((accs[1] + accs[3]) + (accs[5] + accs[7])) + out_v[r, pl.ds(LN, LN)]
        zf = bfb - lfb
        w0a = w0_v[0, pl.ds(0, LN)]
        w0b = w0_v[0, pl.ds(LN, LN)]
        out_v[r, pl.ds(0, LN)] = (acc0 - zf * w0a) / lfb
        out_v[r, pl.ds(LN, LN)] = (acc1 - zf * w0b) / lfb

    fire(0, rows0, sem0)

    def outer(k, _):
        r0 = 2 * k
        fire(r0 + 1, rows1, sem1)
        drain(r0, rows0, sem0)
        accum(r0, rows0)

        @pl.when(k < RPW // 2 - 1)
        def _():
            fire(r0 + 2, rows0, sem0)

        drain(r0 + 1, rows1, sem1)
        accum(r0 + 1, rows1)
        return 0

    lax.fori_loop(0, RPW // 2, outer, 0)
    pltpu.sync_copy(out_v.at[:, :], out_hbm.at[pl.ds(base, RPW), :])


@jax.jit
def kernel(wordids, lens, W):
    mesh = plsc.VectorSubcoreMesh(core_axis_name="c", subcore_axis_name="s")
    f = functools.partial(
        pl.kernel,
        out_type=jax.ShapeDtypeStruct((B, D), jnp.float32),
        mesh=mesh,
        compiler_params=pltpu.CompilerParams(
            needs_layout_passes=False, use_tc_tiling_on_sc=False),
        scratch_types=[
            pltpu.VMEM((RPW, L), jnp.int32),
            pltpu.VMEM((RPW, LN), jnp.float32),
            pltpu.VMEM((RPW, LN), jnp.float32),
            pltpu.VMEM((L, D), jnp.float32),
            pltpu.VMEM((L, D), jnp.float32),
            pltpu.VMEM((RPW, D), jnp.float32),
            pltpu.VMEM((1, D), jnp.float32),
            pltpu.VMEM((RPW,), jnp.int32),
            pltpu.SemaphoreType.DMA,
            pltpu.SemaphoreType.DMA,
        ],
    )(_body)
    return f(wordids, lens, W)


# speedup vs baseline: 1.0439x; 1.0439x over previous
import functools

import jax
import jax.numpy as jnp
from jax import lax
from jax.experimental import pallas as pl
from jax.experimental.pallas import tpu as pltpu
from jax.experimental.pallas import tpu_sc as plsc

B, L, V, D = 4096, 200, 1000000, 32
NC, NS = 2, 16
NW = NC * NS
RPW = B // NW
LN = 16
NBUF = 4


def _body(wid_hbm, lens_hbm, w_hbm, out_hbm,
          wid_v, lensb_v, buckb_v, rows_v, out_v, w0_v, lens_sc,
          sem0, sem1, sem2, sem3):
    sems = (sem0, sem1, sem2, sem3)
    cid = lax.axis_index("c")
    sid = lax.axis_index("s")
    w = cid * NS + sid
    base = w * RPW

    pltpu.sync_copy(wid_hbm.at[pl.ds(base, RPW), :], wid_v.at[:, :])
    pltpu.sync_copy(w_hbm.at[pl.ds(0, 1), :], w0_v)
    pltpu.sync_copy(lens_hbm.at[pl.ds(base, RPW)], lens_sc)

    lane = lax.iota(jnp.int32, LN)
    iotaf = lane.astype(jnp.float32)

    def prep(r):
        r2 = lax.rem(r, LN)
        lv = lens_sc[pl.ds(r - r2, LN)]
        len_i = jnp.max(jnp.where(lane == r2, lv, 0))
        lenf = len_i.astype(jnp.float32)
        bucket = jnp.minimum((len_i + 31) // 32 * 32, L)
        ones = jnp.full((LN,), 1.0, jnp.float32)
        lfb = ones * lenf
        lensb_v[r, :] = lfb
        buckb_v[r, :] = ones * bucket.astype(jnp.float32)
        for c in range(13):
            off = min(c * LN, L - LN)
            ids = wid_v[r, pl.ds(off, LN)]
            posf = iotaf + float(off)
            wid_v[r, pl.ds(off, LN)] = jnp.where(posf < lfb, ids, 0)

    def nbuckets(r):
        return (jnp.max(buckb_v[r, :]).astype(jnp.int32) + 31) // 32

    def fire(r, j):
        rows_buf, sem = rows_v.at[j], sems[j]
        nb = nbuckets(r)

        def fire_block(cc, _):
            pltpu.async_copy(
                w_hbm.at[wid_v.at[r, pl.ds(cc * 32, 32)]],
                rows_buf.at[pl.ds(cc * 32, 32), :], sem)
            return 0

        lax.fori_loop(0, jnp.minimum(nb, 6), fire_block, 0)

        @pl.when(nb >= 7)
        def _():
            pltpu.async_copy(
                w_hbm.at[wid_v.at[r, pl.ds(192, 8)]],
                rows_buf.at[pl.ds(192, 8), :], sem)

    def drain(r, j):
        rows_buf, sem = rows_v.at[j], sems[j]
        nb = nbuckets(r)

        def drain_block(cc, _):
            pltpu.make_async_copy(
                w_hbm.at[wid_v.at[r, pl.ds(0, 32)]],
                rows_buf.at[pl.ds(0, 32), :], sem).wait()
            return 0

        lax.fori_loop(0, jnp.minimum(nb, 6), drain_block, 0)

        @pl.when(nb >= 7)
        def _():
            pltpu.make_async_copy(
                w_hbm.at[wid_v.at[r, pl.ds(192, 8)]],
                rows_buf.at[pl.ds(192, 8), :], sem).wait()

    def accum(r, j):
        rows_buf = rows_v.at[j]
        lfb = lensb_v[r, :]
        bfb = buckb_v[r, :]
        bk = jnp.max(bfb).astype(jnp.int32)
        zero = jnp.zeros((LN,), jnp.float32)

        def block_body(c, accs):
            accs = list(accs)
            b32 = c * 32
            for u in range(32):
                accs[2 * (u % 4)] = accs[2 * (u % 4)] + rows_buf[b32 + u, pl.ds(0, LN)]
                accs[2 * (u % 4) + 1] = accs[2 * (u % 4) + 1] + rows_buf[b32 + u, pl.ds(LN, LN)]
            return tuple(accs)

        accs = lax.fori_loop(0, bk // 32, block_body, (zero,) * 8)
        accs = list(accs)

        @pl.when(bk == L)
        def _():
            a = [rows_buf[192 + u, pl.ds(0, LN)] for u in range(8)]
            b = [rows_buf[192 + u, pl.ds(LN, LN)] for u in range(8)]
            t0 = ((a[0] + a[1]) + (a[2] + a[3])) + ((a[4] + a[5]) + (a[6] + a[7]))
            t1 = ((b[0] + b[1]) + (b[2] + b[3])) + ((b[4] + b[5]) + (b[6] + b[7]))
            out_v[r, pl.ds(0, LN)] = t0
            out_v[r, pl.ds(LN, LN)] = t1

        @pl.when(bk < L)
        def _():
            out_v[r, pl.ds(0, LN)] = zero
            out_v[r, pl.ds(LN, LN)] = zero

        acc0 = ((accs[0] + accs[2]) + (accs[4] + accs[6])) + out_v[r, pl.ds(0, LN)]
        acc1 = ((accs[1] + accs[3]) + (accs[5] + accs[7])) + out_v[r, pl.ds(LN, LN)]
        zf = bfb - lfb
        w0a = w0_v[0, pl.ds(0, LN)]
        w0b = w0_v[0, pl.ds(LN, LN)]
        out_v[r, pl.ds(0, LN)] = (acc0 - zf * w0a) / lfb
        out_v[r, pl.ds(LN, LN)] = (acc1 - zf * w0b) / lfb

    for j in range(NBUF - 1):
        prep(j)
        fire(j, j)

    def outer(k, _):
        r0 = k * NBUF
        for j in range(NBUF):
            r = r0 + j
            rn = r + NBUF - 1

            @pl.when(rn < RPW)
            def _():
                prep(rn)
                fire(rn, (j + NBUF - 1) % NBUF)

            drain(r, j)
            accum(r, j)
        return 0

    lax.fori_loop(0, RPW // NBUF, outer, 0)
    pltpu.sync_copy(out_v.at[:, :], out_hbm.at[pl.ds(base, RPW), :])


@jax.jit
def kernel(wordids, lens, W):
    mesh = plsc.VectorSubcoreMesh(core_axis_name="c", subcore_axis_name="s")
    f = functools.partial(
        pl.kernel,
        out_type=jax.ShapeDtypeStruct((B, D), jnp.float32),
        mesh=mesh,
        compiler_params=pltpu.CompilerParams(
            needs_layout_passes=False, use_tc_tiling_on_sc=False),
        scratch_types=[
            pltpu.VMEM((RPW, L), jnp.int32),
            pltpu.VMEM((RPW, LN), jnp.float32),
            pltpu.VMEM((RPW, LN), jnp.float32),
            pltpu.VMEM((NBUF, L, D), jnp.float32),
            pltpu.VMEM((RPW, D), jnp.float32),
            pltpu.VMEM((1, D), jnp.float32),
            pltpu.VMEM((RPW,), jnp.int32),
            pltpu.SemaphoreType.DMA,
            pltpu.SemaphoreType.DMA,
            pltpu.SemaphoreType.DMA,
            pltpu.SemaphoreType.DMA,
        ],
    )(_body)
    return f(wordids, lens, W)
